# baseline (speedup 1.0000x reference)
import functools

import jax
import jax.numpy as jnp
from jax import lax
from jax.experimental import pallas as pl
from jax.experimental.pallas import tpu as pltpu
from jax.experimental.pallas import tpu_sc as plsc

N_NODES = 10000
H = 512
ROW_BLK = 2000
N_BLOCKS = N_NODES // ROW_BLK

NC = 2
NS = 16
CW = 128
EB = 96
NB = 105
NQ = EB // 4
PT = NB * EB
E_PAD = NS * PT
ZS = 632
SLAB_ROWS = NS * ZS
PAD_DST = SLAB_ROWS - 1
WB = 624
WB_TAIL = N_NODES - NS * WB


def _dense_a_body(x_ref, agg_ref, w1_ref, b1_ref, t_ref, sums_ref):
    C = agg_ref.shape[0]
    agg = jnp.concatenate([agg_ref[c] for c in range(C)], axis=1)
    z = x_ref[...] + agg
    t = jnp.dot(z, w1_ref[...].T, preferred_element_type=jnp.float32) + b1_ref[...]
    t_ref[...] = t
    s1 = jnp.sum(t, axis=0, keepdims=True)
    s2 = jnp.sum(t * t, axis=0, keepdims=True)
    blk = jnp.concatenate([s1, s2], axis=0)

    @pl.when(pl.program_id(0) == 0)
    def _init():
        sums_ref[...] = blk

    @pl.when(pl.program_id(0) != 0)
    def _acc():
        sums_ref[...] = sums_ref[...] + blk


def _dense_a(x, aggc, w1, b1):
    ic = x.shape[1]
    C = ic // CW
    return pl.pallas_call(
        _dense_a_body,
        grid=(N_BLOCKS,),
        in_specs=[
            pl.BlockSpec((ROW_BLK, ic), lambda i: (i, 0)),
            pl.BlockSpec((C, ROW_BLK, CW), lambda i: (0, i, 0)),
            pl.BlockSpec((H, ic), lambda i: (0, 0)),
            pl.BlockSpec((1, H), lambda i: (0, 0)),
        ],
        out_specs=[
            pl.BlockSpec((ROW_BLK, H), lambda i: (i, 0)),
            pl.BlockSpec((2, H), lambda i: (0, 0)),
        ],
        out_shape=[
            jax.ShapeDtypeStruct((N_NODES, H), jnp.float32),
            jax.ShapeDtypeStruct((2, H), jnp.float32),
        ],
    )(x, aggc, w1, b1.reshape(1, H))


def _dense_b_body(t_ref, sums_ref, gb_ref, w2_ref, b2_ref, ben_ref,
                  out_ref, outc_ref, *, bn_eps):
    n = jnp.float32(N_NODES)
    mu = sums_ref[0:1, :] / n
    var = sums_ref[1:2, :] / n - mu * mu
    gamma = gb_ref[0:1, :]
    beta = gb_ref[1:2, :]
    scale = gamma / jnp.sqrt(var + bn_eps)
    u = (t_ref[...] - mu) * scale + beta
    u = jnp.maximum(u, 0.0)
    h = jnp.dot(u, w2_ref[...].T, preferred_element_type=jnp.float32) + b2_ref[...]
    h = jnp.maximum(h, 0.0)
    out_ref[...] = h
    for c in range(H // CW):
        outc_ref[c] = h[:, c * CW:(c + 1) * CW] + ben_ref[0, c * CW:(c + 1) * CW]


def _dense_b(t, sums, gamma, beta, w2, b2, be_next, bn_eps):
    gb = jnp.stack([gamma, beta], axis=0)
    return pl.pallas_call(
        functools.partial(_dense_b_body, bn_eps=bn_eps),
        grid=(N_BLOCKS,),
        in_specs=[
            pl.BlockSpec((ROW_BLK, H), lambda i: (i, 0)),
            pl.BlockSpec((2, H), lambda i: (0, 0)),
            pl.BlockSpec((2, H), lambda i: (0, 0)),
            pl.BlockSpec((H, H), lambda i: (0, 0)),
            pl.BlockSpec((1, H), lambda i: (0, 0)),
            pl.BlockSpec((1, H), lambda i: (0, 0)),
        ],
        out_specs=[
            pl.BlockSpec((ROW_BLK, H), lambda i: (i, 0)),
            pl.BlockSpec((H // CW, ROW_BLK, CW), lambda i: (0, i, 0)),
        ],
        out_shape=[
            jax.ShapeDtypeStruct((N_NODES, H), jnp.float32),
            jax.ShapeDtypeStruct((H // CW, N_NODES, CW), jnp.float32),
        ],
    )(t, sums, gb, w2, b2.reshape(1, H), be_next.reshape(1, H))


def _head_body(h1_ref, h2_ref, h3_ref, h4_ref, wa_ref, wb_ref, wc_ref, wd_ref,
               b1_ref, w2_ref, b2_ref, out_ref, sm_ref):
    bf = jnp.bfloat16
    u = jnp.dot(h1_ref[...].astype(bf), wa_ref[...].astype(bf).T,
                preferred_element_type=jnp.float32)
    u += jnp.dot(h2_ref[...].astype(bf), wb_ref[...].astype(bf).T,
                 preferred_element_type=jnp.float32)
    u += jnp.dot(h3_ref[...].astype(bf), wc_ref[...].astype(bf).T,
                 preferred_element_type=jnp.float32)
    u += jnp.dot(h4_ref[...].astype(bf), wd_ref[...].astype(bf).T,
                 preferred_element_type=jnp.float32)
    u = jnp.maximum(u + b1_ref[...], 0.0)
    v = jnp.dot(u, w2_ref[...].T, preferred_element_type=jnp.float32) + b2_ref[...]
    out_ref[...] = v
    m = jnp.max(v, axis=1, keepdims=True)
    e = jnp.exp(v - m)
    sm_ref[...] = e / jnp.sum(e, axis=1, keepdims=True)


def _head(h1, h2, h3, h4, w1, b1, w2, b2):
    H4 = 4 * H
    OUT = w2.shape[0]
    HEAD_BLK = 1000
    wa, wb, wc, wd = (w1[:, i * H:(i + 1) * H] for i in range(4))
    return pl.pallas_call(
        _head_body,
        grid=(N_NODES // HEAD_BLK,),
        in_specs=[pl.BlockSpec((HEAD_BLK, H), lambda i: (i, 0))] * 4 + [
            pl.BlockSpec((H4, H), lambda i: (0, 0)),
            pl.BlockSpec((H4, H), lambda i: (0, 0)),
            pl.BlockSpec((H4, H), lambda i: (0, 0)),
            pl.BlockSpec((H4, H), lambda i: (0, 0)),
            pl.BlockSpec((1, H4), lambda i: (0, 0)),
            pl.BlockSpec((OUT, H4), lambda i: (0, 0)),
            pl.BlockSpec((1, OUT), lambda i: (0, 0)),
        ],
        out_specs=[
            pl.BlockSpec((HEAD_BLK, OUT), lambda i: (i, 0)),
            pl.BlockSpec((HEAD_BLK, OUT), lambda i: (i, 0)),
        ],
        out_shape=[
            jax.ShapeDtypeStruct((N_NODES, OUT), jnp.float32),
            jax.ShapeDtypeStruct((N_NODES, OUT), jnp.float32),
        ],
    )(h1, h2, h3, h4, wa, wb, wc, wd, b1.reshape(1, H4), w2, b2.reshape(1, OUT))


def _sc_agg_build(C):
    CH = C // NC
    mesh = plsc.VectorSubcoreMesh(core_axis_name="c", subcore_axis_name="s",
                                  num_cores=NC, num_subcores=NS)

    @functools.partial(
        pl.kernel,
        out_type=jax.ShapeDtypeStruct((C, N_NODES, CW), jnp.float32),
        mesh=mesh,
        scratch_types=[
            pltpu.VMEM_SHARED((SLAB_ROWS, CW), jnp.float32),
            [pltpu.VMEM((2, EB), jnp.int32)] * 3,
            [pltpu.VMEM((4, EB), jnp.float32)] * 3,
            [pltpu.VMEM((EB,), jnp.int32)] * 3,
            [pltpu.VMEM((EB, CW), jnp.float32)] * 3,
            pltpu.VMEM((4, CW), jnp.float32),
            [pltpu.SemaphoreType.DMA] * 3,
            [pltpu.SemaphoreType.DMA] * 3,
            [pltpu.SemaphoreType.DMA] * 3,
        ],
    )
    def k(x_hbm, cmb_hbm, ea_hbm, weT_hbm, z_hbm,
          out_hbm, slab, cmbb, eabb, dstb, gbuf, wbuf,
          csem, gsem, ssem):
        kc = lax.axis_index("c")
        s = lax.axis_index("s")

        def issue_stage(chunk, b, j):
            pltpu.async_copy(cmb_hbm.at[chunk, s, b], cmbb[j], csem[j])
            pltpu.async_copy(ea_hbm.at[s, b], eabb[j], csem[j])

        def wait_stage(chunk, j):
            pltpu.make_async_copy(cmb_hbm.at[chunk, s, 0], cmbb[j],
                                  csem[j]).wait()
            pltpu.make_async_copy(ea_hbm.at[s, 0], eabb[j], csem[j]).wait()

        def issue_gather(j):
            pltpu.async_copy(x_hbm.at[cmbb[j].at[0]], gbuf[j], gsem[j])

        def wait_gather(j):
            pltpu.make_async_copy(x_hbm.at[cmbb[j].at[0]], gbuf[j],
                                  gsem[j]).wait()

        def issue_scatter(j):
            pltpu.async_copy(gbuf[j], slab.at[dstb[j]], ssem[j], add=True)

        def wait_scatter(j):
            pltpu.make_async_copy(gbuf[j], slab.at[dstb[j]], ssem[j]).wait()

        def compute(j, wv):
            def quad_body(q, carry2):
                ea16 = eabb[j][q // 6, pl.ds((q % 6) * 16, 16)]
                for d in range(4):
                    a0 = ea16[4 * d + 0]
                    a1 = ea16[4 * d + 1]
                    a2 = ea16[4 * d + 2]
                    a3 = ea16[4 * d + 3]
                    for v in range(CW // 16):
                        g = gbuf[j][4 * q + d, pl.ds(v * 16, 16)]
                        m = ((g + a0 * wv[0][v]) + (a1 * wv[1][v] + a2 * wv[2][v])
                             + a3 * wv[3][v])
                        gbuf[j][4 * q + d, pl.ds(v * 16, 16)] = \
                            jnp.maximum(m, 0.0)
                return carry2

            lax.fori_loop(0, NQ, quad_body, 0)

        def copy_dst(j):
            for kk in range(EB // 16):
                dstb[j][pl.ds(kk * 16, 16)] = cmbb[j][1, pl.ds(kk * 16, 16)]

        def batch_step(chunk, b, j, wv, *, first=False, prefetch=True,
                       more=True):
            wait_gather(j)
            copy_dst(j)
            if not first:
                wait_scatter((j + 1) % 3)
            if more:
                wait_stage(chunk, (j + 1) % 3)
                issue_gather((j + 1) % 3)
            compute(j, wv)
            if prefetch:
                issue_stage(chunk, b + 3, j)
            issue_scatter(j)

        for cc in range(CH):
            chunk = kc * CH + cc
            pltpu.sync_copy(weT_hbm.at[chunk], wbuf)
            pltpu.sync_copy(z_hbm, slab.at[pl.ds(s * ZS, ZS)])
            plsc.subcore_barrier()
            wv = [[wbuf[jj, pl.ds(v * 16, 16)] for v in range(CW // 16)]
                  for jj in range(4)]

            for j in range(3):
                issue_stage(chunk, j, j)
            wait_stage(chunk, 0)
            issue_gather(0)
            batch_step(chunk, 0, 0, wv, first=True)
            batch_step(chunk, 1, 1, wv, first=True)
            batch_step(chunk, 2, 2, wv)

            def ring_body(gg, carry):
                b = 3 * gg
                batch_step(chunk, b + 0, 0, wv)
                batch_step(chunk, b + 1, 1, wv)
                batch_step(chunk, b + 2, 2, wv)
                return carry

            lax.fori_loop(1, NB // 3 - 1, ring_body, 0)
            batch_step(chunk, NB - 3, 0, wv, prefetch=False)
            batch_step(chunk, NB - 2, 1, wv, prefetch=False)
            batch_step(chunk, NB - 1, 2, wv, prefetch=False, more=False)
            wait_scatter(1)
            wait_scatter(2)
            plsc.subcore_barrier()
            pltpu.sync_copy(slab.at[pl.ds(s * WB, WB)],
                            out_hbm.at[chunk, pl.ds(s * WB, WB)])

            @pl.when(s == 0)
            def _tail():
                pltpu.sync_copy(slab.at[pl.ds(NS * WB, WB_TAIL)],
                                out_hbm.at[chunk, pl.ds(NS * WB, WB_TAIL)])

            plsc.subcore_barrier()

    return k


def _edge_prep(src, dst, edge_attr):
    pad = E_PAD - src.shape[0]
    src_p = jnp.concatenate([src, jnp.zeros((pad,), jnp.int32)])
    dst_p = jnp.concatenate([dst, jnp.full((pad,), PAD_DST, jnp.int32)])
    ea_p = jnp.concatenate([edge_attr, jnp.zeros((pad, 4), jnp.float32)])
    dst_r = dst_p.reshape(NS, NB, 1, EB)
    ea_r = ea_p.reshape(NS, NB, 4, EB)
    out = {}
    for C in (2, 4):
        off = (jnp.arange(C, dtype=jnp.int32) * N_NODES)[:, None]
        src_off = (src_p[None, :] + off).reshape(C, NS, NB, 1, EB)
        dst_b = jnp.broadcast_to(dst_r[None], (C, NS, NB, 1, EB))
        out[C] = jnp.concatenate([src_off, dst_b], axis=3)
    return out, ea_r, jnp.zeros((ZS, CW), jnp.float32)


def _sc_agg(xflat, cmb, ea_r, zeros_strip, we, sc_kernels):
    C = xflat.shape[0] // N_NODES
    weT_c = we.T.reshape(4, C, CW).transpose(1, 0, 2)
    return sc_kernels[C](xflat, cmb, ea_r, weT_c, zeros_strip)


def _gine_layer(x, xflat, eprep, p, be_next, bn_eps, sc_kernels):
    cmb_all, ea_r, zstrip = eprep
    C = xflat.shape[0] // N_NODES
    aggc = _sc_agg(xflat, cmb_all[C], ea_r, zstrip, p['We'], sc_kernels)
    t, sums = _dense_a(x, aggc, p['W1'], p['b1'])
    h, hc = _dense_b(t, sums, p['gamma'], p['beta'], p['W2'], p['b2'],
                     be_next, bn_eps)
    return h, hc.reshape(4 * N_NODES, CW)


def kernel(x, edge_index, edge_attr, params):
    bn_eps = float(H)
    src = edge_index[0]
    dst = edge_index[1]
    eprep = _edge_prep(src, dst, edge_attr)
    sc_kernels = {2: _sc_agg_build(2), 4: _sc_agg_build(4)}
    be1 = params['conv1']['be']
    be2 = params['conv2']['be']
    be3 = params['conv3']['be']
    xflat = ((x + be1[None, :]).reshape(N_NODES, 2, CW)
             .transpose(1, 0, 2).reshape(2 * N_NODES, CW))
    h1, f1 = _gine_layer(x, xflat, eprep, params['conv1'], be2, bn_eps,
                         sc_kernels)
    h2, f2 = _gine_layer(h1, f1, eprep, params['conv2'], be3, bn_eps,
                         sc_kernels)
    h3, f3 = _gine_layer(h2, f2, eprep, params['conv3'], be3, bn_eps,
                         sc_kernels)
    h4, _ = _gine_layer(h3, f3, eprep, params['conv3'], be3, bn_eps,
                        sc_kernels)
    return _head(h1, h2, h3, h4, params['lin1']['W'], params['lin1']['b'],
                 params['lin2']['W'], params['lin2']['b'])


# speedup vs baseline: 1.0002x; 1.0002x over previous
import functools

import jax
import jax.numpy as jnp
from jax import lax
from jax.experimental import pallas as pl
from jax.experimental.pallas import tpu as pltpu
from jax.experimental.pallas import tpu_sc as plsc

N_NODES = 10000
H = 512
ROW_BLK = 2000
N_BLOCKS = N_NODES // ROW_BLK

NC = 2
NS = 16
CW = 128
EB = 96
NB = 105
NQ = EB // 4
PT = NB * EB
E_PAD = NS * PT
ZS = 632
SLAB_ROWS = NS * ZS
PAD_DST = SLAB_ROWS - 1
WB = 624
WB_TAIL = N_NODES - NS * WB


def _dense_a_body(x_ref, agg_ref, w1_ref, b1_ref, t_ref, sums_ref):
    C = agg_ref.shape[0]
    agg = jnp.concatenate([agg_ref[c] for c in range(C)], axis=1)
    z = x_ref[...] + agg
    t = jnp.dot(z, w1_ref[...].T, preferred_element_type=jnp.float32) + b1_ref[...]
    t_ref[...] = t
    s1 = jnp.sum(t, axis=0, keepdims=True)
    s2 = jnp.sum(t * t, axis=0, keepdims=True)
    blk = jnp.concatenate([s1, s2], axis=0)

    @pl.when(pl.program_id(0) == 0)
    def _init():
        sums_ref[...] = blk

    @pl.when(pl.program_id(0) != 0)
    def _acc():
        sums_ref[...] = sums_ref[...] + blk


def _dense_a(x, aggc, w1, b1):
    ic = x.shape[1]
    C = ic // CW
    return pl.pallas_call(
        _dense_a_body,
        grid=(N_BLOCKS,),
        in_specs=[
            pl.BlockSpec((ROW_BLK, ic), lambda i: (i, 0)),
            pl.BlockSpec((C, ROW_BLK, CW), lambda i: (0, i, 0)),
            pl.BlockSpec((H, ic), lambda i: (0, 0)),
            pl.BlockSpec((1, H), lambda i: (0, 0)),
        ],
        out_specs=[
            pl.BlockSpec((ROW_BLK, H), lambda i: (i, 0)),
            pl.BlockSpec((2, H), lambda i: (0, 0)),
        ],
        out_shape=[
            jax.ShapeDtypeStruct((N_NODES, H), jnp.float32),
            jax.ShapeDtypeStruct((2, H), jnp.float32),
        ],
    )(x, aggc, w1, b1.reshape(1, H))


def _dense_b_body(t_ref, sums_ref, gb_ref, w2_ref, b2_ref, ben_ref,
                  out_ref, outc_ref, *, bn_eps):
    n = jnp.float32(N_NODES)
    mu = sums_ref[0:1, :] / n
    var = sums_ref[1:2, :] / n - mu * mu
    gamma = gb_ref[0:1, :]
    beta = gb_ref[1:2, :]
    scale = gamma / jnp.sqrt(var + bn_eps)
    u = (t_ref[...] - mu) * scale + beta
    u = jnp.maximum(u, 0.0)
    h = jnp.dot(u, w2_ref[...].T, preferred_element_type=jnp.float32) + b2_ref[...]
    h = jnp.maximum(h, 0.0)
    out_ref[...] = h
    for c in range(H // CW):
        outc_ref[c] = h[:, c * CW:(c + 1) * CW] + ben_ref[0, c * CW:(c + 1) * CW]


def _dense_b(t, sums, gamma, beta, w2, b2, be_next, bn_eps):
    gb = jnp.stack([gamma, beta], axis=0)
    return pl.pallas_call(
        functools.partial(_dense_b_body, bn_eps=bn_eps),
        grid=(N_BLOCKS,),
        in_specs=[
            pl.BlockSpec((ROW_BLK, H), lambda i: (i, 0)),
            pl.BlockSpec((2, H), lambda i: (0, 0)),
            pl.BlockSpec((2, H), lambda i: (0, 0)),
            pl.BlockSpec((H, H), lambda i: (0, 0)),
            pl.BlockSpec((1, H), lambda i: (0, 0)),
            pl.BlockSpec((1, H), lambda i: (0, 0)),
        ],
        out_specs=[
            pl.BlockSpec((ROW_BLK, H), lambda i: (i, 0)),
            pl.BlockSpec((H // CW, ROW_BLK, CW), lambda i: (0, i, 0)),
        ],
        out_shape=[
            jax.ShapeDtypeStruct((N_NODES, H), jnp.float32),
            jax.ShapeDtypeStruct((H // CW, N_NODES, CW), jnp.float32),
        ],
    )(t, sums, gb, w2, b2.reshape(1, H), be_next.reshape(1, H))


def _head_body(h1_ref, h2_ref, h3_ref, h4_ref, wa_ref, wb_ref, wc_ref, wd_ref,
               b1_ref, w2_ref, b2_ref, out_ref, sm_ref):
    u = jnp.dot(h1_ref[...], wa_ref[...].T, preferred_element_type=jnp.float32)
    u += jnp.dot(h2_ref[...], wb_ref[...].T, preferred_element_type=jnp.float32)
    u += jnp.dot(h3_ref[...], wc_ref[...].T, preferred_element_type=jnp.float32)
    u += jnp.dot(h4_ref[...], wd_ref[...].T, preferred_element_type=jnp.float32)
    u = jnp.maximum(u + b1_ref[...], 0.0)
    v = jnp.dot(u, w2_ref[...].T, preferred_element_type=jnp.float32) + b2_ref[...]
    out_ref[...] = v
    m = jnp.max(v, axis=1, keepdims=True)
    e = jnp.exp(v - m)
    sm_ref[...] = e / jnp.sum(e, axis=1, keepdims=True)


def _head(h1, h2, h3, h4, w1, b1, w2, b2):
    H4 = 4 * H
    OUT = w2.shape[0]
    HEAD_BLK = 1000
    wa, wb, wc, wd = (w1[:, i * H:(i + 1) * H] for i in range(4))
    return pl.pallas_call(
        _head_body,
        grid=(N_NODES // HEAD_BLK,),
        in_specs=[pl.BlockSpec((HEAD_BLK, H), lambda i: (i, 0))] * 4 + [
            pl.BlockSpec((H4, H), lambda i: (0, 0)),
            pl.BlockSpec((H4, H), lambda i: (0, 0)),
            pl.BlockSpec((H4, H), lambda i: (0, 0)),
            pl.BlockSpec((H4, H), lambda i: (0, 0)),
            pl.BlockSpec((1, H4), lambda i: (0, 0)),
            pl.BlockSpec((OUT, H4), lambda i: (0, 0)),
            pl.BlockSpec((1, OUT), lambda i: (0, 0)),
        ],
        out_specs=[
            pl.BlockSpec((HEAD_BLK, OUT), lambda i: (i, 0)),
            pl.BlockSpec((HEAD_BLK, OUT), lambda i: (i, 0)),
        ],
        out_shape=[
            jax.ShapeDtypeStruct((N_NODES, OUT), jnp.float32),
            jax.ShapeDtypeStruct((N_NODES, OUT), jnp.float32),
        ],
    )(h1, h2, h3, h4, wa, wb, wc, wd, b1.reshape(1, H4), w2, b2.reshape(1, OUT))


def _sc_agg_build(C):
    CH = C // NC
    mesh = plsc.VectorSubcoreMesh(core_axis_name="c", subcore_axis_name="s",
                                  num_cores=NC, num_subcores=NS)

    @functools.partial(
        pl.kernel,
        out_type=jax.ShapeDtypeStruct((C, N_NODES, CW), jnp.float32),
        mesh=mesh,
        scratch_types=[
            pltpu.VMEM_SHARED((SLAB_ROWS, CW), jnp.float32),
            [pltpu.VMEM((2, EB), jnp.int32)] * 3,
            [pltpu.VMEM((4, EB), jnp.float32)] * 3,
            [pltpu.VMEM((EB,), jnp.int32)] * 3,
            [pltpu.VMEM((EB, CW), jnp.float32)] * 3,
            pltpu.VMEM((4, CW), jnp.float32),
            [pltpu.SemaphoreType.DMA] * 3,
            [pltpu.SemaphoreType.DMA] * 3,
            [pltpu.SemaphoreType.DMA] * 3,
        ],
    )
    def k(x_hbm, cmb_hbm, ea_hbm, weT_hbm, z_hbm,
          out_hbm, slab, cmbb, eabb, dstb, gbuf, wbuf,
          csem, gsem, ssem):
        kc = lax.axis_index("c")
        s = lax.axis_index("s")

        def issue_stage(chunk, b, j):
            pltpu.async_copy(cmb_hbm.at[chunk, s, b], cmbb[j], csem[j])
            pltpu.async_copy(ea_hbm.at[s, b], eabb[j], csem[j])

        def wait_stage(chunk, j):
            pltpu.make_async_copy(cmb_hbm.at[chunk, s, 0], cmbb[j],
                                  csem[j]).wait()
            pltpu.make_async_copy(ea_hbm.at[s, 0], eabb[j], csem[j]).wait()

        def issue_gather(j):
            pltpu.async_copy(x_hbm.at[cmbb[j].at[0]], gbuf[j], gsem[j])

        def wait_gather(j):
            pltpu.make_async_copy(x_hbm.at[cmbb[j].at[0]], gbuf[j],
                                  gsem[j]).wait()

        def issue_scatter(j):
            pltpu.async_copy(gbuf[j], slab.at[dstb[j]], ssem[j], add=True)

        def wait_scatter(j):
            pltpu.make_async_copy(gbuf[j], slab.at[dstb[j]], ssem[j]).wait()

        def compute(j, wv):
            def quad_body(q, carry2):
                ea16 = eabb[j][q // 6, pl.ds((q % 6) * 16, 16)]
                for d in range(4):
                    a0 = ea16[4 * d + 0]
                    a1 = ea16[4 * d + 1]
                    a2 = ea16[4 * d + 2]
                    a3 = ea16[4 * d + 3]
                    for v in range(CW // 16):
                        g = gbuf[j][4 * q + d, pl.ds(v * 16, 16)]
                        m = ((g + a0 * wv[0][v]) + (a1 * wv[1][v] + a2 * wv[2][v])
                             + a3 * wv[3][v])
                        gbuf[j][4 * q + d, pl.ds(v * 16, 16)] = \
                            jnp.maximum(m, 0.0)
                return carry2

            lax.fori_loop(0, NQ, quad_body, 0)

        def copy_dst(j):
            for kk in range(EB // 16):
                dstb[j][pl.ds(kk * 16, 16)] = cmbb[j][1, pl.ds(kk * 16, 16)]

        def batch_step(chunk, b, j, wv, *, first=False, prefetch=True,
                       more=True):
            wait_gather(j)
            copy_dst(j)
            if not first:
                wait_scatter((j + 1) % 3)
            if more:
                wait_stage(chunk, (j + 1) % 3)
                issue_gather((j + 1) % 3)
            compute(j, wv)
            if prefetch:
                issue_stage(chunk, b + 3, j)
            issue_scatter(j)

        for cc in range(CH):
            chunk = kc * CH + cc
            pltpu.sync_copy(weT_hbm.at[chunk], wbuf)
            pltpu.sync_copy(z_hbm, slab.at[pl.ds(s * ZS, ZS)])
            plsc.subcore_barrier()
            wv = [[wbuf[jj, pl.ds(v * 16, 16)] for v in range(CW // 16)]
                  for jj in range(4)]

            for j in range(3):
                issue_stage(chunk, j, j)
            wait_stage(chunk, 0)
            issue_gather(0)
            batch_step(chunk, 0, 0, wv, first=True)
            batch_step(chunk, 1, 1, wv, first=True)
            batch_step(chunk, 2, 2, wv)

            def ring_body(gg, carry):
                b = 3 * gg
                batch_step(chunk, b + 0, 0, wv)
                batch_step(chunk, b + 1, 1, wv)
                batch_step(chunk, b + 2, 2, wv)
                return carry

            lax.fori_loop(1, NB // 3 - 1, ring_body, 0)
            batch_step(chunk, NB - 3, 0, wv, prefetch=False)
            batch_step(chunk, NB - 2, 1, wv, prefetch=False)
            batch_step(chunk, NB - 1, 2, wv, prefetch=False, more=False)
            wait_scatter(1)
            wait_scatter(2)
            plsc.subcore_barrier()
            pltpu.sync_copy(slab.at[pl.ds(s * WB, WB)],
                            out_hbm.at[chunk, pl.ds(s * WB, WB)])

            @pl.when(s == 0)
            def _tail():
                pltpu.sync_copy(slab.at[pl.ds(NS * WB, WB_TAIL)],
                                out_hbm.at[chunk, pl.ds(NS * WB, WB_TAIL)])

            plsc.subcore_barrier()

    return k


def _edge_prep(src, dst, edge_attr):
    pad = E_PAD - src.shape[0]
    src_p = jnp.concatenate([src, jnp.zeros((pad,), jnp.int32)])
    dst_p = jnp.concatenate([dst, jnp.full((pad,), PAD_DST, jnp.int32)])
    ea_p = jnp.concatenate([edge_attr, jnp.zeros((pad, 4), jnp.float32)])
    dst_r = dst_p.reshape(NS, NB, 1, EB)
    ea_r = ea_p.reshape(NS, NB, 4, EB)
    out = {}
    for C in (2, 4):
        off = (jnp.arange(C, dtype=jnp.int32) * N_NODES)[:, None]
        src_off = (src_p[None, :] + off).reshape(C, NS, NB, 1, EB)
        dst_b = jnp.broadcast_to(dst_r[None], (C, NS, NB, 1, EB))
        out[C] = jnp.concatenate([src_off, dst_b], axis=3)
    return out, ea_r, jnp.zeros((ZS, CW), jnp.float32)


def _sc_agg(xflat, cmb, ea_r, zeros_strip, we, sc_kernels):
    C = xflat.shape[0] // N_NODES
    weT_c = we.T.reshape(4, C, CW).transpose(1, 0, 2)
    return sc_kernels[C](xflat, cmb, ea_r, weT_c, zeros_strip)


def _gine_layer(x, xflat, eprep, p, be_next, bn_eps, sc_kernels):
    cmb_all, ea_r, zstrip = eprep
    C = xflat.shape[0] // N_NODES
    aggc = _sc_agg(xflat, cmb_all[C], ea_r, zstrip, p['We'], sc_kernels)
    t, sums = _dense_a(x, aggc, p['W1'], p['b1'])
    h, hc = _dense_b(t, sums, p['gamma'], p['beta'], p['W2'], p['b2'],
                     be_next, bn_eps)
    return h, hc.reshape(4 * N_NODES, CW)


def kernel(x, edge_index, edge_attr, params):
    bn_eps = float(H)
    src = edge_index[0]
    dst = edge_index[1]
    eprep = _edge_prep(src, dst, edge_attr)
    sc_kernels = {2: _sc_agg_build(2), 4: _sc_agg_build(4)}
    be1 = params['conv1']['be']
    be2 = params['conv2']['be']
    be3 = params['conv3']['be']
    xflat = ((x + be1[None, :]).reshape(N_NODES, 2, CW)
             .transpose(1, 0, 2).reshape(2 * N_NODES, CW))
    h1, f1 = _gine_layer(x, xflat, eprep, params['conv1'], be2, bn_eps,
                         sc_kernels)
    h2, f2 = _gine_layer(h1, f1, eprep, params['conv2'], be3, bn_eps,
                         sc_kernels)
    h3, f3 = _gine_layer(h2, f2, eprep, params['conv3'], be3, bn_eps,
                         sc_kernels)
    h4, _ = _gine_layer(h3, f3, eprep, params['conv3'], be3, bn_eps,
                        sc_kernels)
    return _head(h1, h2, h3, h4, params['lin1']['W'], params['lin1']['b'],
                 params['lin2']['W'], params['lin2']['b'])
